# baseline (speedup 1.0000x reference)
import functools

import jax
import jax.numpy as jnp
from jax import lax
from jax.experimental import pallas as pl
from jax.experimental.pallas import tpu as pltpu
from jax.experimental.pallas import tpu_sc as plsc

_N = 10000
_E = 320000
_D_HID = 128
_D_OUT = 64
_CW = 8

_NP = 10000
_NC = 2
_NS = 16
_NW = _NC * _NS
_EPW = _E // _NW
_CH = 100
_NCH = _EPW // _CH


def _make_propagate(dp):
  rpt = _NP // _NS

  mesh = plsc.VectorSubcoreMesh(core_axis_name="c", subcore_axis_name="s")

  @functools.partial(
      pl.kernel,
      mesh=mesh,
      compiler_params=pltpu.CompilerParams(use_tc_tiling_on_sc=False),
      out_type=jax.ShapeDtypeStruct((_NC, _NP, dp), jnp.float32),
      scratch_types=[
          pltpu.VMEM((_NCH, _CH), jnp.int32),
          pltpu.VMEM((_CH,), jnp.int32),
          pltpu.VMEM((_CH,), jnp.int32),
          pltpu.VMEM((_CH, dp), jnp.float32),
          pltpu.VMEM((_CH, dp), jnp.float32),
          pltpu.VMEM_SHARED((_NP, dp), jnp.float32),
          pltpu.SemaphoreType.DMA,
          pltpu.SemaphoreType.DMA,
      ],
  )
  def prop(vals, ed, out, src_l, dst0, dst1, rows0, rows1, accum, sem0, sem1):
    c = lax.axis_index("c")
    s = lax.axis_index("s")
    w = s * _NC + c

    idx_cp = pltpu.async_copy(ed.at[0, w], src_l, sem0)

    def zrow(i, carry):
      for j in range(dp // 16):
        rows0[i, pl.ds(j * 16, 16)] = jnp.zeros((16,), jnp.float32)
      return carry

    lax.fori_loop(0, _CH, zrow, 0)
    rbase = s * rpt
    nz, zrem = rpt // _CH, rpt % _CH
    for k in range(nz):
      pltpu.sync_copy(rows0, accum.at[pl.ds(rbase + k * _CH, _CH)])
    if zrem:
      pltpu.sync_copy(rows0.at[pl.ds(0, zrem)],
                      accum.at[pl.ds(rbase + nz * _CH, zrem)])
    idx_cp.wait()
    plsc.subcore_barrier()

    pltpu.async_copy(ed.at[1, w, 0], dst0, sem0)
    pltpu.async_copy(vals.at[src_l.at[0]], rows0, sem0)
    pltpu.async_copy(ed.at[1, w, 1], dst1, sem1)
    pltpu.async_copy(vals.at[src_l.at[1]], rows1, sem1)

    def pair(i, carry):
      j0 = 2 * i
      pltpu.make_async_copy(ed.at[1, w, 0], dst0, sem0).wait()
      pltpu.make_async_copy(vals.at[src_l.at[0]], rows0, sem0).wait()
      pltpu.sync_copy(rows0, accum.at[dst0], add=True)
      n0 = jnp.minimum(j0 + 2, _NCH - 1)
      pltpu.async_copy(ed.at[1, w, n0], dst0, sem0)
      pltpu.async_copy(vals.at[src_l.at[n0]], rows0, sem0)
      j1 = j0 + 1
      pltpu.make_async_copy(ed.at[1, w, 1], dst1, sem1).wait()
      pltpu.make_async_copy(vals.at[src_l.at[1]], rows1, sem1).wait()
      pltpu.sync_copy(rows1, accum.at[dst1], add=True)
      n1 = jnp.minimum(j1 + 2, _NCH - 1)
      pltpu.async_copy(ed.at[1, w, n1], dst1, sem1)
      pltpu.async_copy(vals.at[src_l.at[n1]], rows1, sem1)
      return carry

    lax.fori_loop(0, _NCH // 2, pair, 0)
    pltpu.make_async_copy(ed.at[1, w, 0], dst0, sem0).wait()
    pltpu.make_async_copy(vals.at[src_l.at[0]], rows0, sem0).wait()
    pltpu.make_async_copy(ed.at[1, w, 1], dst1, sem1).wait()
    pltpu.make_async_copy(vals.at[src_l.at[1]], rows1, sem1).wait()
    plsc.subcore_barrier()

    pltpu.sync_copy(accum.at[pl.ds(rbase, rpt)], out.at[c, pl.ds(rbase, rpt)])

  return prop


def _make_count():
  rpt = _NP // _NS

  mesh = plsc.VectorSubcoreMesh(core_axis_name="c", subcore_axis_name="s")

  @functools.partial(
      pl.kernel,
      mesh=mesh,
      compiler_params=pltpu.CompilerParams(use_tc_tiling_on_sc=False),
      out_type=jax.ShapeDtypeStruct((_NC, _NP, _CW), jnp.float32),
      scratch_types=[
          pltpu.VMEM((_CH,), jnp.int32),
          pltpu.VMEM((_CH,), jnp.int32),
          pltpu.VMEM((_CH, _CW), jnp.float32),
          pltpu.VMEM_SHARED((_NP, _CW), jnp.float32),
          pltpu.SemaphoreType.DMA,
          pltpu.SemaphoreType.DMA,
      ],
  )
  def count(ed, cnt, dst0, dst1, ones, cacc, sem0, sem1):
    c = lax.axis_index("c")
    s = lax.axis_index("s")
    w = s * _NC + c

    def zrow(i, carry):
      ones[i, :] = jnp.zeros((_CW,), jnp.float32)
      return carry

    lax.fori_loop(0, _CH, zrow, 0)
    rbase = s * rpt
    nz, zrem = rpt // _CH, rpt % _CH
    for k in range(nz):
      pltpu.sync_copy(ones, cacc.at[pl.ds(rbase + k * _CH, _CH)])
    if zrem:
      pltpu.sync_copy(ones.at[pl.ds(0, zrem)],
                      cacc.at[pl.ds(rbase + nz * _CH, zrem)])

    def onerow(i, carry):
      ones[i, :] = jnp.full((_CW,), 1.0, jnp.float32)
      return carry

    lax.fori_loop(0, _CH, onerow, 0)
    plsc.subcore_barrier()

    pltpu.async_copy(ed.at[1, w, 0], dst0, sem0)
    pltpu.async_copy(ed.at[1, w, 1], dst1, sem1)

    def pair(i, carry):
      j0 = 2 * i
      pltpu.make_async_copy(ed.at[1, w, 0], dst0, sem0).wait()
      pltpu.sync_copy(ones, cacc.at[dst0], add=True)
      pltpu.async_copy(ed.at[1, w, jnp.minimum(j0 + 2, _NCH - 1)], dst0, sem0)
      j1 = j0 + 1
      pltpu.make_async_copy(ed.at[1, w, 1], dst1, sem1).wait()
      pltpu.sync_copy(ones, cacc.at[dst1], add=True)
      pltpu.async_copy(ed.at[1, w, jnp.minimum(j1 + 2, _NCH - 1)], dst1, sem1)
      return carry

    lax.fori_loop(0, _NCH // 2, pair, 0)
    pltpu.make_async_copy(ed.at[1, w, 0], dst0, sem0).wait()
    pltpu.make_async_copy(ed.at[1, w, 1], dst1, sem1).wait()
    plsc.subcore_barrier()

    pltpu.sync_copy(cacc.at[pl.ds(rbase, rpt)], cnt.at[c, pl.ds(rbase, rpt)])

  return count


_PROP_CACHE = {}


def _get_prop(dp):
  if dp not in _PROP_CACHE:
    _PROP_CACHE[dp] = _make_propagate(dp)
  return _PROP_CACHE[dp]


def _get_count():
  if "count" not in _PROP_CACHE:
    _PROP_CACHE["count"] = _make_count()
  return _PROP_CACHE["count"]


def _stage_a(x_ref, w_ref, out_ref):
  out_ref[...] = jnp.dot(x_ref[...], w_ref[...],
                         preferred_element_type=jnp.float32)


def _stage_b(p_ref, c_ref, b1_ref, g1_ref, be1_ref, wl_ref,
             h1_ref, g_ref, invc_ref, dinv_ref):
  p = p_ref[0] + p_ref[1]
  cnt = c_ref[0, :, :1] + c_ref[1, :, :1]
  t = p + b1_ref[...]
  nrm = jnp.sqrt(jnp.sum(t * t, axis=1, keepdims=True))
  t = t / jnp.maximum(nrm, 1e-12)
  m = jnp.mean(t, axis=0, keepdims=True)
  v = jnp.mean((t - m) ** 2, axis=0, keepdims=True)
  t = g1_ref[...] * (t - m) / jnp.sqrt(v + 1e-5) + be1_ref[...]
  t = jnp.maximum(t, 0.0)
  h1_ref[...] = t
  g_ref[...] = jnp.dot(t, wl_ref[...], preferred_element_type=jnp.float32)
  invc_ref[...] = 1.0 / jnp.maximum(cnt, 1.0)
  dinv_ref[...] = lax.rsqrt(cnt + 1.0)


def _stage_c0(h1_ref, wr_ref, st_ref):
  st_ref[...] = jnp.dot(h1_ref[...], wr_ref[...],
                        preferred_element_type=jnp.float32)


def _stage_c(p_ref, invc_ref, st_ref, bl_ref, g2_ref, be2_ref,
             wg_ref, dinv_ref, xw_ref, xs_ref):
  p = p_ref[0] + p_ref[1]
  t = p * invc_ref[...] + bl_ref[...] + st_ref[...]
  m = jnp.mean(t, axis=0, keepdims=True)
  v = jnp.mean((t - m) ** 2, axis=0, keepdims=True)
  t = g2_ref[...] * (t - m) / jnp.sqrt(v + 1e-5) + be2_ref[...]
  t = jnp.maximum(t, 0.0)
  xw = jnp.dot(t, wg_ref[...], preferred_element_type=jnp.float32)
  xw_ref[...] = xw
  xs_ref[...] = xw * dinv_ref[...]


def _stage_d(p_ref, dinv_ref, xw_ref, bg_ref, out_ref):
  agg = p_ref[0] + p_ref[1]
  d = dinv_ref[...]
  o = d * agg + (d * d) * xw_ref[...] + bg_ref[...]
  mx = jnp.max(o, axis=1, keepdims=True)
  z = o - mx
  lse = jnp.log(jnp.sum(jnp.exp(z), axis=1, keepdims=True))
  out_ref[...] = z - lse


def kernel(x, edge_index, lin1_W, lin1_b, bn1_g, bn1_b, sage_Wl, sage_bl,
           sage_Wr, bn2_g, bn2_b, gcn_W, gcn_b):
  ed = edge_index.astype(jnp.int32).reshape(2, _NW, _NCH, _CH)
  b1 = lin1_b.reshape(1, -1)
  g1 = bn1_g.reshape(1, -1)
  be1 = bn1_b.reshape(1, -1)
  bl = sage_bl.reshape(1, -1)
  g2 = bn2_g.reshape(1, -1)
  be2 = bn2_b.reshape(1, -1)
  bg = gcn_b.reshape(1, -1)
  f32 = jnp.float32

  h0 = pl.pallas_call(
      _stage_a, out_shape=jax.ShapeDtypeStruct((_N, _D_HID), f32),
  )(x, lin1_W)

  c0 = _get_count()(ed)
  p0 = _get_prop(_D_HID)(h0, ed)

  h1, g, invc, dinv = pl.pallas_call(
      _stage_b,
      out_shape=[
          jax.ShapeDtypeStruct((_N, _D_HID), f32),
          jax.ShapeDtypeStruct((_N, _D_HID), f32),
          jax.ShapeDtypeStruct((_N, 1), f32),
          jax.ShapeDtypeStruct((_N, 1), f32),
      ],
  )(p0, c0, b1, g1, be1, sage_Wl)

  p1 = _get_prop(_D_HID)(g, ed)

  selfterm = pl.pallas_call(
      _stage_c0, out_shape=jax.ShapeDtypeStruct((_N, _D_HID), f32),
  )(h1, sage_Wr)

  xw, xs = pl.pallas_call(
      _stage_c,
      out_shape=[
          jax.ShapeDtypeStruct((_N, _D_OUT), f32),
          jax.ShapeDtypeStruct((_N, _D_OUT), f32),
      ],
  )(p1, invc, selfterm, bl, g2, be2, gcn_W, dinv)

  p2 = _get_prop(_D_OUT)(xs, ed)

  out = pl.pallas_call(
      _stage_d, out_shape=jax.ShapeDtypeStruct((_N, _D_OUT), f32),
  )(p2, dinv, xw, bg)
  return out


# speedup vs baseline: 1.0438x; 1.0438x over previous
import functools

import jax
import jax.numpy as jnp
from jax import lax
from jax.experimental import pallas as pl
from jax.experimental.pallas import tpu as pltpu
from jax.experimental.pallas import tpu_sc as plsc

_N = 10000
_E = 320000
_D_HID = 128
_D_OUT = 64
_CW = 8

_NP = 10000
_NC = 2
_NS = 16
_NW = _NC * _NS
_EPW = _E // _NW
_CH = 100
_NCH = _EPW // _CH


def _make_propagate(dp, with_counts=False):
  rpt = _NP // _NS

  mesh = plsc.VectorSubcoreMesh(core_axis_name="c", subcore_axis_name="s")

  main_out = jax.ShapeDtypeStruct((_NC, _NP, dp), jnp.float32)
  cnt_out = jax.ShapeDtypeStruct((_NC, _NP, _CW), jnp.float32)
  scratch = [
      pltpu.VMEM((_NCH, _CH), jnp.int32),
      pltpu.VMEM((_CH,), jnp.int32),
      pltpu.VMEM((_CH,), jnp.int32),
      pltpu.VMEM((_CH, dp), jnp.float32),
      pltpu.VMEM((_CH, dp), jnp.float32),
      pltpu.VMEM_SHARED((_NP, dp), jnp.float32),
      pltpu.SemaphoreType.DMA,
      pltpu.SemaphoreType.DMA,
  ]
  if with_counts:
    scratch += [
        pltpu.VMEM((_CH, _CW), jnp.float32),
        pltpu.VMEM_SHARED((_NP, _CW), jnp.float32),
    ]

  @functools.partial(
      pl.kernel,
      mesh=mesh,
      compiler_params=pltpu.CompilerParams(use_tc_tiling_on_sc=False),
      out_type=[main_out, cnt_out] if with_counts else main_out,
      scratch_types=scratch,
  )
  def prop(vals, ed, *rest):
    if with_counts:
      (out, cnt, src_l, dst0, dst1, rows0, rows1, accum, sem0, sem1,
       ones, cacc) = rest
    else:
      out, src_l, dst0, dst1, rows0, rows1, accum, sem0, sem1 = rest
    c = lax.axis_index("c")
    s = lax.axis_index("s")
    w = s * _NC + c

    idx_cp = pltpu.async_copy(ed.at[0, w], src_l, sem0)

    def zrow(i, carry):
      for j in range(dp // 16):
        rows0[i, pl.ds(j * 16, 16)] = jnp.zeros((16,), jnp.float32)
      if with_counts:
        ones[i, :] = jnp.zeros((_CW,), jnp.float32)
      return carry

    lax.fori_loop(0, _CH, zrow, 0)
    rbase = s * rpt
    nz, zrem = rpt // _CH, rpt % _CH
    for k in range(nz):
      pltpu.sync_copy(rows0, accum.at[pl.ds(rbase + k * _CH, _CH)])
      if with_counts:
        pltpu.sync_copy(ones, cacc.at[pl.ds(rbase + k * _CH, _CH)])
    if zrem:
      pltpu.sync_copy(rows0.at[pl.ds(0, zrem)],
                      accum.at[pl.ds(rbase + nz * _CH, zrem)])
      if with_counts:
        pltpu.sync_copy(ones.at[pl.ds(0, zrem)],
                        cacc.at[pl.ds(rbase + nz * _CH, zrem)])
    if with_counts:
      def onerow(i, carry):
        ones[i, :] = jnp.full((_CW,), 1.0, jnp.float32)
        return carry
      lax.fori_loop(0, _CH, onerow, 0)
    idx_cp.wait()
    plsc.subcore_barrier()

    pltpu.async_copy(ed.at[1, w, 0], dst0, sem0)
    pltpu.async_copy(vals.at[src_l.at[0]], rows0, sem0)
    pltpu.async_copy(ed.at[1, w, 1], dst1, sem1)
    pltpu.async_copy(vals.at[src_l.at[1]], rows1, sem1)

    def pair(i, carry):
      j0 = 2 * i
      pltpu.make_async_copy(ed.at[1, w, 0], dst0, sem0).wait()
      pltpu.make_async_copy(vals.at[src_l.at[0]], rows0, sem0).wait()
      pltpu.sync_copy(rows0, accum.at[dst0], add=True)
      if with_counts:
        pltpu.sync_copy(ones, cacc.at[dst0], add=True)
      n0 = jnp.minimum(j0 + 2, _NCH - 1)
      pltpu.async_copy(ed.at[1, w, n0], dst0, sem0)
      pltpu.async_copy(vals.at[src_l.at[n0]], rows0, sem0)
      j1 = j0 + 1
      pltpu.make_async_copy(ed.at[1, w, 1], dst1, sem1).wait()
      pltpu.make_async_copy(vals.at[src_l.at[1]], rows1, sem1).wait()
      pltpu.sync_copy(rows1, accum.at[dst1], add=True)
      if with_counts:
        pltpu.sync_copy(ones, cacc.at[dst1], add=True)
      n1 = jnp.minimum(j1 + 2, _NCH - 1)
      pltpu.async_copy(ed.at[1, w, n1], dst1, sem1)
      pltpu.async_copy(vals.at[src_l.at[n1]], rows1, sem1)
      return carry

    lax.fori_loop(0, _NCH // 2, pair, 0)
    pltpu.make_async_copy(ed.at[1, w, 0], dst0, sem0).wait()
    pltpu.make_async_copy(vals.at[src_l.at[0]], rows0, sem0).wait()
    pltpu.make_async_copy(ed.at[1, w, 1], dst1, sem1).wait()
    pltpu.make_async_copy(vals.at[src_l.at[1]], rows1, sem1).wait()
    plsc.subcore_barrier()

    pltpu.sync_copy(accum.at[pl.ds(rbase, rpt)], out.at[c, pl.ds(rbase, rpt)])
    if with_counts:
      pltpu.sync_copy(cacc.at[pl.ds(rbase, rpt)],
                      cnt.at[c, pl.ds(rbase, rpt)])

  return prop


_PROP_CACHE = {}


def _get_prop(dp, with_counts=False):
  key = (dp, with_counts)
  if key not in _PROP_CACHE:
    _PROP_CACHE[key] = _make_propagate(dp, with_counts)
  return _PROP_CACHE[key]


def _stage_a(x_ref, w_ref, out_ref):
  out_ref[...] = jnp.dot(x_ref[...], w_ref[...],
                         preferred_element_type=jnp.float32)


def _stage_b(p_ref, c_ref, b1_ref, g1_ref, be1_ref, wl_ref,
             h1_ref, g_ref, invc_ref, dinv_ref):
  p = p_ref[0] + p_ref[1]
  cnt = c_ref[0, :, :1] + c_ref[1, :, :1]
  t = p + b1_ref[...]
  nrm = jnp.sqrt(jnp.sum(t * t, axis=1, keepdims=True))
  t = t / jnp.maximum(nrm, 1e-12)
  m = jnp.mean(t, axis=0, keepdims=True)
  v = jnp.mean((t - m) ** 2, axis=0, keepdims=True)
  t = g1_ref[...] * (t - m) / jnp.sqrt(v + 1e-5) + be1_ref[...]
  t = jnp.maximum(t, 0.0)
  h1_ref[...] = t
  g_ref[...] = jnp.dot(t, wl_ref[...], preferred_element_type=jnp.float32)
  invc_ref[...] = 1.0 / jnp.maximum(cnt, 1.0)
  dinv_ref[...] = lax.rsqrt(cnt + 1.0)


def _stage_c0(h1_ref, wr_ref, st_ref):
  st_ref[...] = jnp.dot(h1_ref[...], wr_ref[...],
                        preferred_element_type=jnp.float32)


def _stage_c(p_ref, invc_ref, st_ref, bl_ref, g2_ref, be2_ref,
             wg_ref, dinv_ref, xw_ref, xs_ref):
  p = p_ref[0] + p_ref[1]
  t = p * invc_ref[...] + bl_ref[...] + st_ref[...]
  m = jnp.mean(t, axis=0, keepdims=True)
  v = jnp.mean((t - m) ** 2, axis=0, keepdims=True)
  t = g2_ref[...] * (t - m) / jnp.sqrt(v + 1e-5) + be2_ref[...]
  t = jnp.maximum(t, 0.0)
  xw = jnp.dot(t, wg_ref[...], preferred_element_type=jnp.float32)
  xw_ref[...] = xw
  xs_ref[...] = xw * dinv_ref[...]


def _stage_d(p_ref, dinv_ref, xw_ref, bg_ref, out_ref):
  agg = p_ref[0] + p_ref[1]
  d = dinv_ref[...]
  o = d * agg + (d * d) * xw_ref[...] + bg_ref[...]
  mx = jnp.max(o, axis=1, keepdims=True)
  z = o - mx
  lse = jnp.log(jnp.sum(jnp.exp(z), axis=1, keepdims=True))
  out_ref[...] = z - lse


def kernel(x, edge_index, lin1_W, lin1_b, bn1_g, bn1_b, sage_Wl, sage_bl,
           sage_Wr, bn2_g, bn2_b, gcn_W, gcn_b):
  ed = edge_index.astype(jnp.int32).reshape(2, _NW, _NCH, _CH)
  b1 = lin1_b.reshape(1, -1)
  g1 = bn1_g.reshape(1, -1)
  be1 = bn1_b.reshape(1, -1)
  bl = sage_bl.reshape(1, -1)
  g2 = bn2_g.reshape(1, -1)
  be2 = bn2_b.reshape(1, -1)
  bg = gcn_b.reshape(1, -1)
  f32 = jnp.float32

  h0 = pl.pallas_call(
      _stage_a, out_shape=jax.ShapeDtypeStruct((_N, _D_HID), f32),
  )(x, lin1_W)

  p0, c0 = _get_prop(_D_HID, with_counts=True)(h0, ed)

  h1, g, invc, dinv = pl.pallas_call(
      _stage_b,
      out_shape=[
          jax.ShapeDtypeStruct((_N, _D_HID), f32),
          jax.ShapeDtypeStruct((_N, _D_HID), f32),
          jax.ShapeDtypeStruct((_N, 1), f32),
          jax.ShapeDtypeStruct((_N, 1), f32),
      ],
  )(p0, c0, b1, g1, be1, sage_Wl)

  p1 = _get_prop(_D_HID)(g, ed)

  selfterm = pl.pallas_call(
      _stage_c0, out_shape=jax.ShapeDtypeStruct((_N, _D_HID), f32),
  )(h1, sage_Wr)

  xw, xs = pl.pallas_call(
      _stage_c,
      out_shape=[
          jax.ShapeDtypeStruct((_N, _D_OUT), f32),
          jax.ShapeDtypeStruct((_N, _D_OUT), f32),
      ],
  )(p1, invc, selfterm, bl, g2, be2, gcn_W, dinv)

  p2 = _get_prop(_D_OUT)(xs, ed)

  out = pl.pallas_call(
      _stage_d, out_shape=jax.ShapeDtypeStruct((_N, _D_OUT), f32),
  )(p2, dinv, xw, bg)
  return out
